# baseline (speedup 1.0000x reference)
import jax
import jax.numpy as jnp
from jax.experimental import pallas as pl
from jax.experimental.pallas import tpu as pltpu

_FEAT = 128
_W0_OFF, _W1_OFF, _W2_OFF, _BIAS_OFF = 0, 32, 160, 288
_LR, _LE = 20, 4


def _mlp_kernel(inv_ref, reg_ref, exp_ref, slab_ref, qr_ref, qe_ref, *, t):
    f32 = jnp.float32
    bias = slab_ref[_BIAS_OFF:_BIAS_OFF + 8, :]
    tail = 32
    zrow = jnp.zeros((1, _FEAT), f32)
    w0a = jnp.concatenate(
        [jnp.broadcast_to(zrow, (tail - 1, _FEAT)), slab_ref[0:1, :]], axis=0)
    w0b = jnp.concatenate(
        [jnp.broadcast_to(zrow, (tail - _LR, _FEAT)),
         slab_ref[_W0_OFF + 1:_W0_OFF + 1 + _LR, :]], axis=0)
    w0c = jnp.concatenate(
        [jnp.broadcast_to(zrow, (tail - _LE, _FEAT)),
         slab_ref[_W0_OFF + 1 + _LR:_W0_OFF + 1 + _LR + _LE, :]], axis=0)
    h = jnp.dot(inv_ref[:, t - tail:t], w0a, preferred_element_type=f32)
    h = h + jnp.dot(reg_ref[:, t - tail:t], w0b, preferred_element_type=f32)
    h = h + jnp.dot(exp_ref[:, t - tail:t], w0c, preferred_element_type=f32)
    h = jnp.maximum(h + bias[0:1, :], 0.0)
    h = jnp.dot(h, slab_ref[_W1_OFF:_W1_OFF + _FEAT, :],
                preferred_element_type=f32)
    h = jnp.maximum(h + bias[1:2, :], 0.0)
    h = jnp.dot(h, slab_ref[_W2_OFF:_W2_OFF + _FEAT, 0:8],
                preferred_element_type=f32)
    h = jnp.maximum(h + bias[2:3, 0:8], 0.0)
    h = jnp.floor(h)
    qr_ref[...] = h[:, 0:1]
    qe_ref[...] = h[:, 1:2]


def _pick_bm(b):
    for bm in (2048, 1024, 512, 256, 128, 64, 32, 16, 8):
        if b % bm == 0:
            return bm
    return b


@jax.jit
def _run(slab, inv, reg, exp):
    B, T = inv.shape
    bm = _pick_bm(B)
    nb = B // bm
    n_rows = slab.shape[0]
    kern = lambda *refs: _mlp_kernel(*refs, t=T)
    qr, qe = pl.pallas_call(
        kern,
        out_shape=[jax.ShapeDtypeStruct((B, 1), jnp.float32),
                   jax.ShapeDtypeStruct((B, 1), jnp.float32)],
        grid=(nb,),
        in_specs=[
            pl.BlockSpec((bm, T), lambda i: (i, 0)),
            pl.BlockSpec((bm, T), lambda i: (i, 0)),
            pl.BlockSpec((bm, T), lambda i: (i, 0)),
            pl.BlockSpec((n_rows, _FEAT), lambda i: (0, 0)),
        ],
        out_specs=[pl.BlockSpec((bm, 1), lambda i: (i, 0)),
                   pl.BlockSpec((bm, 1), lambda i: (i, 0))],
        compiler_params=pltpu.CompilerParams(
            dimension_semantics=("parallel",)),
    )(inv, reg, exp, slab)
    return qr, qe


def kernel(slab, past_demands, past_inventories, past_regular_orders,
           past_expedited_orders, past_costs):
    del past_demands, past_costs
    inv = jnp.asarray(past_inventories, jnp.float32)
    reg = jnp.asarray(past_regular_orders, jnp.float32)
    exp = jnp.asarray(past_expedited_orders, jnp.float32)
    B, T = inv.shape[0], inv.shape[1]
    qr, qe = _run(slab, inv.reshape(B, T), reg.reshape(B, T),
                  exp.reshape(B, T))
    return qr, qe, None


# speedup vs baseline: 1.0206x; 1.0206x over previous
import jax
import jax.numpy as jnp
from jax.experimental import pallas as pl
from jax.experimental.pallas import tpu as pltpu

_FEAT = 128
_W0_OFF, _W1_OFF, _W2_OFF, _BIAS_OFF = 0, 32, 160, 288
_LR, _LE = 20, 4


def _mlp_kernel(inv_ref, reg_ref, exp_ref, slab_ref, qr_ref, qe_ref, *, t):
    f32 = jnp.float32
    bias = slab_ref[_BIAS_OFF:_BIAS_OFF + 8, :]
    zrow = jnp.zeros((1, _FEAT), f32)
    w0a = jnp.concatenate(
        [jnp.broadcast_to(zrow, (t - 1, _FEAT)), slab_ref[0:1, :]], axis=0)
    w0b = jnp.concatenate(
        [jnp.broadcast_to(zrow, (t - _LR, _FEAT)),
         slab_ref[_W0_OFF + 1:_W0_OFF + 1 + _LR, :]], axis=0)
    w0c = jnp.concatenate(
        [jnp.broadcast_to(zrow, (t - _LE, _FEAT)),
         slab_ref[_W0_OFF + 1 + _LR:_W0_OFF + 1 + _LR + _LE, :]], axis=0)
    h = jnp.dot(inv_ref[...], w0a, preferred_element_type=f32)
    h = h + jnp.dot(reg_ref[...], w0b, preferred_element_type=f32)
    h = h + jnp.dot(exp_ref[...], w0c, preferred_element_type=f32)
    h = jnp.maximum(h + bias[0:1, :], 0.0)
    h = jnp.dot(h, slab_ref[_W1_OFF:_W1_OFF + _FEAT, :],
                preferred_element_type=f32)
    h = jnp.maximum(h + bias[1:2, :], 0.0)
    h = jnp.dot(h, slab_ref[_W2_OFF:_W2_OFF + _FEAT, 0:8],
                preferred_element_type=f32)
    h = jnp.maximum(h + bias[2:3, 0:8], 0.0)
    h = jnp.floor(h)
    qr_ref[...] = h[:, 0:1]
    qe_ref[...] = h[:, 1:2]


def _pick_bm(b):
    for bm in (8192, 4096, 2048, 1024, 512, 256, 128, 64, 32, 16, 8):
        if b % bm == 0:
            return bm
    return b


@jax.jit
def _run(slab, inv, reg, exp):
    B, T = inv.shape
    bm = _pick_bm(B)
    nb = B // bm
    n_rows = slab.shape[0]
    kern = lambda *refs: _mlp_kernel(*refs, t=T)
    qr, qe = pl.pallas_call(
        kern,
        out_shape=[jax.ShapeDtypeStruct((B, 1), jnp.float32),
                   jax.ShapeDtypeStruct((B, 1), jnp.float32)],
        grid=(nb,),
        in_specs=[
            pl.BlockSpec((bm, T), lambda i: (i, 0)),
            pl.BlockSpec((bm, T), lambda i: (i, 0)),
            pl.BlockSpec((bm, T), lambda i: (i, 0)),
            pl.BlockSpec((n_rows, _FEAT), lambda i: (0, 0)),
        ],
        out_specs=[pl.BlockSpec((bm, 1), lambda i: (i, 0)),
                   pl.BlockSpec((bm, 1), lambda i: (i, 0))],
        compiler_params=pltpu.CompilerParams(
            dimension_semantics=("parallel",)),
    )(inv, reg, exp, slab)
    return qr, qe


def kernel(slab, past_demands, past_inventories, past_regular_orders,
           past_expedited_orders, past_costs):
    del past_demands, past_costs
    inv = jnp.asarray(past_inventories, jnp.float32)
    reg = jnp.asarray(past_regular_orders, jnp.float32)
    exp = jnp.asarray(past_expedited_orders, jnp.float32)
    B, T = inv.shape[0], inv.shape[1]
    qr, qe = _run(slab, inv.reshape(B, T), reg.reshape(B, T),
                  exp.reshape(B, T))
    return qr, qe, None


# speedup vs baseline: 1.0218x; 1.0012x over previous
import jax
import jax.numpy as jnp
from jax.experimental import pallas as pl
from jax.experimental.pallas import tpu as pltpu

_FEAT = 128
_W0_OFF, _W1_OFF, _W2_OFF, _BIAS_OFF = 0, 32, 160, 288
_LR, _LE = 20, 4


def _mlp_kernel(inv_ref, reg_ref, exp_ref, slab_ref, qr_ref, qe_ref, *, t):
    f32 = jnp.float32
    bias = slab_ref[_BIAS_OFF:_BIAS_OFF + 8, :]
    zrow = jnp.zeros((1, _FEAT), f32)
    w0a = jnp.concatenate(
        [jnp.broadcast_to(zrow, (t - 1, _FEAT)), slab_ref[0:1, :]], axis=0)
    w0b = jnp.concatenate(
        [jnp.broadcast_to(zrow, (t - _LR, _FEAT)),
         slab_ref[_W0_OFF + 1:_W0_OFF + 1 + _LR, :]], axis=0)
    w0c = jnp.concatenate(
        [jnp.broadcast_to(zrow, (t - _LE, _FEAT)),
         slab_ref[_W0_OFF + 1 + _LR:_W0_OFF + 1 + _LR + _LE, :]], axis=0)
    h = jnp.dot(inv_ref[...], w0a, preferred_element_type=f32)
    h = h + jnp.dot(reg_ref[...], w0b, preferred_element_type=f32)
    h = h + jnp.dot(exp_ref[...], w0c, preferred_element_type=f32)
    h = jnp.maximum(h + bias[0:1, :], 0.0)
    h = jnp.dot(h, slab_ref[_W1_OFF:_W1_OFF + _FEAT, :],
                preferred_element_type=f32)
    h = jnp.maximum(h + bias[1:2, :], 0.0)
    h = jnp.dot(h, slab_ref[_W2_OFF:_W2_OFF + _FEAT, 0:8],
                preferred_element_type=f32)
    h = jnp.maximum(h + bias[2:3, 0:8], 0.0)
    h = jnp.floor(h)
    qr_ref[...] = h[:, 0:1]
    qe_ref[...] = h[:, 1:2]


def _pick_bm(b):
    for bm in (2048, 1024, 512, 256, 128, 64, 32, 16, 8):
        if b % bm == 0:
            return bm
    return b


@jax.jit
def _run(slab, inv, reg, exp):
    B, T = inv.shape
    bm = _pick_bm(B)
    nb = B // bm
    n_rows = slab.shape[0]
    kern = lambda *refs: _mlp_kernel(*refs, t=T)
    qr, qe = pl.pallas_call(
        kern,
        out_shape=[jax.ShapeDtypeStruct((B, 1), jnp.float32),
                   jax.ShapeDtypeStruct((B, 1), jnp.float32)],
        grid=(nb,),
        in_specs=[
            pl.BlockSpec((bm, T), lambda i: (i, 0)),
            pl.BlockSpec((bm, T), lambda i: (i, 0)),
            pl.BlockSpec((bm, T), lambda i: (i, 0)),
            pl.BlockSpec((n_rows, _FEAT), lambda i: (0, 0)),
        ],
        out_specs=[pl.BlockSpec((bm, 1), lambda i: (i, 0)),
                   pl.BlockSpec((bm, 1), lambda i: (i, 0))],
        compiler_params=pltpu.CompilerParams(
            dimension_semantics=("parallel",)),
    )(inv, reg, exp, slab)
    return qr, qe


def kernel(slab, past_demands, past_inventories, past_regular_orders,
           past_expedited_orders, past_costs):
    del past_demands, past_costs
    inv = jnp.asarray(past_inventories, jnp.float32)
    reg = jnp.asarray(past_regular_orders, jnp.float32)
    exp = jnp.asarray(past_expedited_orders, jnp.float32)
    B, T = inv.shape[0], inv.shape[1]
    qr, qe = _run(slab, inv.reshape(B, T), reg.reshape(B, T),
                  exp.reshape(B, T))
    return qr, qe, None
